# baseline (speedup 1.0000x reference)
import functools

import jax
import jax.numpy as jnp
from jax import lax
from jax.experimental import pallas as pl
from jax.experimental.pallas import tpu as pltpu
from jax.experimental.pallas import tpu_sc as plsc

N = 10000
E = 160000
D = 256
DE = 16
G = 8
CC = 2 * D + DE
MD = CC // 2
HID = 2 * CC
GI = D + MD
GH = 2 * GI
MSGW = 384

E_PAD = 163840
PADN = E_PAD - E
EH = E_PAD // 2

B_PER_W = EH // 32
G_CH = 64
G_NCH = B_PER_W // G_CH

EPT = EH // 16
S_CH = 128
S_NCH = EPT // S_CH
NH = N // 2
SPROWS = 5120
ZROWS = SPROWS // 16

BE = 1024
NB = 1000
PN = 1000
PE = E_PAD // 10

_MESH = plsc.VectorSubcoreMesh(core_axis_name="c", subcore_axis_name="s",
                               num_cores=2, num_subcores=16)

F32 = jnp.float32
BF16 = jnp.bfloat16


G_NPAIR = G_NCH // 2


NSP = 10240


def _gather_body(x_hbm, src_hbm, dst_hbm, xs_hbm, xd_hbm, xsp, idx_s, idx_d,
                 bsa, bsb, bda, bdb,
                 sem_sa, sem_sb, sem_da, sem_db,
                 sem_osa, sem_osb, sem_oda, sem_odb):
    wid = lax.axis_index("s") * 2 + lax.axis_index("c")
    s = lax.axis_index("s")
    base = wid * B_PER_W

    pltpu.sync_copy(x_hbm.at[pl.ds(s * (NSP // 16), NSP // 16)],
                    xsp.at[pl.ds(s * (NSP // 16), NSP // 16)])
    pltpu.sync_copy(src_hbm.at[pl.ds(base, B_PER_W)], idx_s)
    pltpu.sync_copy(dst_hbm.at[pl.ds(base, B_PER_W)], idx_d)
    plsc.subcore_barrier()

    def stream(idx_v, out_hbm, buf_a, buf_b, sem_a, sem_b, sem_oa, sem_ob):
        def g_src(c):
            return xsp.at[idx_v.at[pl.ds(c * G_CH, G_CH)]]

        def o_dst(c):
            return out_hbm.at[pl.ds(base + c * G_CH, G_CH)]

        def prologue():
            pltpu.async_copy(g_src(0), buf_a, sem_a)

        def pair(p):
            c0 = 2 * p
            c1 = c0 + 1

            @pl.when(p > 0)
            def _():
                pltpu.make_async_copy(buf_b, o_dst(c1 - 2), sem_ob).wait()

            pltpu.async_copy(g_src(c1), buf_b, sem_b)
            pltpu.make_async_copy(g_src(c0), buf_a, sem_a).wait()
            pltpu.async_copy(buf_a, o_dst(c0), sem_oa)

            @pl.when(p < G_NPAIR - 1)
            def _():
                pltpu.make_async_copy(buf_a, o_dst(c0), sem_oa).wait()
                pltpu.async_copy(g_src(c0 + 2), buf_a, sem_a)

            pltpu.make_async_copy(g_src(c1), buf_b, sem_b).wait()
            pltpu.async_copy(buf_b, o_dst(c1), sem_ob)

        def epilogue():
            pltpu.make_async_copy(buf_a, o_dst(G_NCH - 2), sem_oa).wait()
            pltpu.make_async_copy(buf_b, o_dst(G_NCH - 1), sem_ob).wait()

        return prologue, pair, epilogue

    streams = [stream(idx_s, xs_hbm, bsa, bsb, sem_sa, sem_sb,
                      sem_osa, sem_osb),
               stream(idx_d, xd_hbm, bda, bdb, sem_da, sem_db,
                      sem_oda, sem_odb)]
    for pro, _, _ in streams:
        pro()

    def pair_both(p, carry):
        for _, pair, _ in streams:
            pair(p)
        return carry

    lax.fori_loop(0, G_NPAIR, pair_both, 0)
    for _, _, epi in streams:
        epi()


def _sc_gather(xi, src_p, dst_p):
    fn = pl.kernel(
        _gather_body,
        out_type=[jax.ShapeDtypeStruct((EH, 128), jnp.int32),
                  jax.ShapeDtypeStruct((EH, 128), jnp.int32)],
        mesh=_MESH,
        scratch_types=[pltpu.VMEM_SHARED((NSP, 128), jnp.int32),
                       pltpu.VMEM((B_PER_W,), jnp.int32),
                       pltpu.VMEM((B_PER_W,), jnp.int32)]
        + [pltpu.VMEM((G_CH, 128), jnp.int32)] * 4
        + [pltpu.SemaphoreType.DMA] * 8,
    )
    return fn(xi, src_p, dst_p)


S_NHALF = S_NCH // 2
S_NPAIR = S_NHALF // 2


def _make_scatter_body(width):
    def body(msg_hbm, dst_hbm, zrow_hbm, out0_hbm, out1_hbm, out2_hbm,
             shared, idx_all, idx2d, buf_a, buf_b, buf_c, buf_d,
             sem_a, sem_b, sem_c, sem_d, sem_sa, sem_sb, sem_sc, sem_sd):
        c = lax.axis_index("c")
        s = lax.axis_index("s")
        base_node = c * NH
        tbase = s * EPT

        pltpu.sync_copy(dst_hbm.at[pl.ds(tbase, EPT)], idx_all)

        def grp(k, carry):
            v = idx_all[pl.ds(k * 16, 16)]
            inr = (v >= base_node) & (v < base_node + NH)
            lv = jnp.where(inr, v - base_node, NH)
            idx2d[k // 8, pl.ds((k % 8) * 16, 16)] = lv
            return carry

        lax.fori_loop(0, EPT // 16, grp, 0)

        def stream(col0, off, buf_a, buf_b, sem_a, sem_b, sem_sa, sem_sb):
            def m_src(cc):
                return msg_hbm.at[pl.ds(tbase + (off + cc) * S_CH, S_CH),
                                  pl.ds(col0, width)]

            def sc_dst(cc):
                return shared.at[idx2d.at[off + cc]]

            def prologue():
                pltpu.async_copy(m_src(0), buf_a, sem_a)

            def pair(p):
                c0 = 2 * p
                c1 = c0 + 1

                @pl.when(p > 0)
                def _():
                    pltpu.make_async_copy(buf_b, sc_dst(c1 - 2),
                                          sem_sb).wait()

                pltpu.async_copy(m_src(c1), buf_b, sem_b)
                pltpu.make_async_copy(m_src(c0), buf_a, sem_a).wait()
                pltpu.async_copy(buf_a, sc_dst(c0), sem_sa, add=True)

                @pl.when(p < S_NPAIR - 1)
                def _():
                    pltpu.make_async_copy(buf_a, sc_dst(c0), sem_sa).wait()
                    pltpu.async_copy(m_src(c0 + 2), buf_a, sem_a)

                pltpu.make_async_copy(m_src(c1), buf_b, sem_b).wait()
                pltpu.async_copy(buf_b, sc_dst(c1), sem_sb, add=True)

            def epilogue():
                pltpu.make_async_copy(buf_a, sc_dst(S_NHALF - 2),
                                      sem_sa).wait()
                pltpu.make_async_copy(buf_b, sc_dst(S_NHALF - 1),
                                      sem_sb).wait()

            return prologue, pair, epilogue

        for col0, out_hbm in ((0, out0_hbm), (128, out1_hbm),
                              (256, out2_hbm)):
            pltpu.sync_copy(zrow_hbm, shared.at[pl.ds(s * ZROWS, ZROWS)])
            plsc.subcore_barrier()

            streams = [stream(col0, 0, buf_a, buf_b, sem_a, sem_b,
                              sem_sa, sem_sb),
                       stream(col0, S_NHALF, buf_c, buf_d, sem_c, sem_d,
                              sem_sc, sem_sd)]
            for pro, _, _ in streams:
                pro()

            def pair_both(p, carry):
                for _, pair, _ in streams:
                    pair(p)
                return carry

            lax.fori_loop(0, S_NPAIR, pair_both, 0)
            for _, _, epi in streams:
                epi()
            plsc.subcore_barrier()

            pltpu.sync_copy(shared.at[pl.ds(s * ZROWS, ZROWS)],
                            out_hbm.at[pl.ds(c * SPROWS + s * ZROWS,
                                             ZROWS)])

    return body


def _sc_scatter(msg, dst_s):
    width = 128
    zrow = jnp.zeros((ZROWS, width), F32)
    fn = pl.kernel(
        _make_scatter_body(width),
        out_type=[jax.ShapeDtypeStruct((2 * SPROWS, width), F32)] * 3,
        mesh=_MESH,
        scratch_types=[pltpu.VMEM_SHARED((SPROWS, width), F32),
                       pltpu.VMEM((EPT,), jnp.int32),
                       pltpu.VMEM((S_NCH, S_CH), jnp.int32)]
        + [pltpu.VMEM((S_CH, width), F32)] * 4
        + [pltpu.SemaphoreType.DMA] * 8,
    )
    return fn(msg, dst_s, zrow)


def _edge_mlp_body(xs_ref, xd_ref, ea_ref,
                   c1x, c1c, cb1, c2, cb2, c3, cb3,
                   p1x, p1c, pb1, p2, pb2, p3, pb3,
                   eo_ref, msg_ref):
    def unpack(v):
        lo = jax.lax.bitcast_convert_type(v << 16, F32).astype(BF16)
        hi = jax.lax.bitcast_convert_type(
            v & jnp.int32(-65536), F32).astype(BF16)
        return jnp.concatenate([lo, hi], axis=1)

    xs = unpack(xs_ref[...])
    xd = unpack(xd_ref[...])
    ea = ea_ref[...].astype(BF16)

    def dot(a, b):
        return jnp.dot(a, b, preferred_element_type=F32)

    cat = jnp.concatenate([xs, xd, ea], axis=1)
    h = dot(cat, jnp.concatenate([c1x[...], c1c[...]])) + cb1[...]
    h = jnp.maximum(h, 0.0).astype(BF16)
    h = dot(h, c2[...]) + cb2[...]
    h = jnp.maximum(h, 0.0).astype(BF16)
    eo = dot(h, c3[...]) + cb3[...]
    eo_ref[...] = eo

    cat2 = jnp.concatenate([xs, xd, eo.astype(BF16)], axis=1)
    g = dot(cat2, jnp.concatenate([p1x[...], p1c[...]])) + pb1[...]
    g = jnp.maximum(g, 0.0).astype(BF16)
    g = dot(g, p2[...]) + pb2[...]
    g = jnp.maximum(g, 0.0).astype(BF16)
    msg_ref[...] = dot(g, p3[...]) + pb3[...]


def _edge_mlp(xs, xd, ea, cw, pw):
    grid = (EH // BE,)

    def blk(r, ccols):
        return pl.BlockSpec((r, ccols), lambda i: (i, 0))

    def full(a):
        return pl.BlockSpec(a.shape, lambda i: tuple(0 for _ in a.shape))

    ins = [blk(BE, 128), blk(BE, 128), blk(BE, DE)]
    args = [xs, xd, ea]
    for w in cw + pw:
        ins.append(full(w))
        args.append(w)
    out = pl.pallas_call(
        _edge_mlp_body,
        grid=grid,
        in_specs=ins,
        out_specs=[blk(BE, DE), blk(BE, MSGW)],
        out_shape=[jax.ShapeDtypeStruct((EH, DE), F32),
                   jax.ShapeDtypeStruct((EH, MSGW), F32)],
    )(*args)
    return out


def _gamma_body(s0a_ref, s0b_ref, s1a_ref, s1b_ref, s2a_ref, s2b_ref, x_ref,
                g1a, g1b, g1c, g1d, gb1, g2, gb2, g3, gb3, out_ref):
    s2 = s2a_ref[...] + s2b_ref[...]
    inv = 1.0 / jnp.maximum(s2[:, MD - D:MD - D + 1], 1.0)
    a0 = ((s0a_ref[...] + s0b_ref[...]) * inv).astype(BF16)
    a1 = ((s1a_ref[...] + s1b_ref[...]) * inv).astype(BF16)
    a2 = (s2[:, :MD - D] * inv).astype(BF16)
    x = x_ref[...].astype(BF16)

    def dot(a, b):
        return jnp.dot(a, b[...], preferred_element_type=F32)

    h = (dot(a0, g1a) + dot(a1, g1b) + dot(a2, g1c) + dot(x, g1d)
         + gb1[...])
    h = jnp.maximum(h, 0.0).astype(BF16)
    h = dot(h, g2) + gb2[...]
    h = jnp.maximum(h, 0.0).astype(BF16)
    out_ref[...] = (dot(h, g3) + gb3[...]).astype(out_ref.dtype)


def _gamma_mlp(slabs_a, slabs_b, xf, gw, out_dtype):
    grid = (N // NB,)

    def blk(r, ccols):
        return pl.BlockSpec((r, ccols), lambda i: (i, 0))

    def full(a):
        return pl.BlockSpec(a.shape, lambda i: (0, 0))

    ins = [blk(NB, 128)] * 6 + [blk(NB, D)]
    args = [slabs_a[0], slabs_b[0], slabs_a[1], slabs_b[1],
            slabs_a[2], slabs_b[2], xf]
    for w in gw:
        ins.append(full(w))
        args.append(w)
    return pl.pallas_call(
        _gamma_body,
        grid=grid,
        in_specs=ins,
        out_specs=blk(NB, D),
        out_shape=jax.ShapeDtypeStruct((N, D), out_dtype),
    )(*args)


def _pool_body(x2_ref, nbf_ref, ptr_ref, ea_ref, ebf_ref,
               aw1, ab1, aw2, ab2, coeff_ref, np_acc, ep_acc, ec_acc):
    i = pl.program_id(0)

    @pl.when(i == 0)
    def _():
        np_acc[...] = jnp.zeros_like(np_acc)
        ep_acc[...] = jnp.zeros_like(ep_acc)
        ec_acc[...] = jnp.zeros_like(ec_acc)

    hi = jax.lax.Precision.HIGHEST
    nbf = nbf_ref[...]
    gids = lax.broadcasted_iota(jnp.int32, (PN, G), 1).astype(F32)
    onehot = (nbf == gids).astype(F32)
    ptrsel = jnp.sum(onehot * ptr_ref[...], axis=1, keepdims=True)
    rowid = (lax.broadcasted_iota(jnp.int32, (PN, 1), 0)
             + i * PN).astype(F32)
    weight = rowid - ptrsel + 1.0
    xw = x2_ref[...] * weight
    np_acc[...] += lax.dot_general(onehot, xw, (((0,), (0,)), ((), ())),
                                   precision=hi)
    ebf = ebf_ref[...]
    gide = lax.broadcasted_iota(jnp.int32, (PE, G), 1).astype(F32)
    onehote = (ebf == gide).astype(F32)
    ep_acc[...] += lax.dot_general(onehote, ea_ref[...],
                                   (((0,), (0,)), ((), ())), precision=hi)
    ones = jnp.ones((PE, 1), F32)
    ec_acc[...] += lax.dot_general(onehote, ones, (((0,), (0,)), ((), ())),
                                   precision=hi)

    @pl.when(i == pl.num_programs(0) - 1)
    def _():
        epool = ep_acc[...] / jnp.maximum(ec_acc[...], 1.0)
        pc = jnp.concatenate([np_acc[...], epool], axis=1)
        h = jnp.maximum(
            jnp.dot(pc, aw1[...], precision=hi,
                    preferred_element_type=F32) + ab1[...], 0.0)
        coeff_ref[...] = jnp.dot(h, aw2[...], precision=hi,
                                 preferred_element_type=F32) + ab2[...]


def _pooling(x2, nbf, ptr8, ea2, ebf, aw1, ab1, aw2, ab2):
    grid = (N // PN,)

    def blk(r, ccols):
        return pl.BlockSpec((r, ccols), lambda i: (i, 0))

    def full(a):
        return pl.BlockSpec(a.shape, lambda i: (0, 0))

    return pl.pallas_call(
        _pool_body,
        grid=grid,
        in_specs=[blk(PN, D), blk(PN, 1), full(ptr8), blk(PE, DE), blk(PE, 1),
                  full(aw1), full(ab1), full(aw2), full(ab2)],
        out_specs=full(jnp.zeros((G, 5))),
        out_shape=jax.ShapeDtypeStruct((G, 5), F32),
        scratch_shapes=[pltpu.VMEM((G, D), F32), pltpu.VMEM((G, DE), F32),
                        pltpu.VMEM((G, 1), F32)],
    )(x2, nbf, ptr8, ea2, ebf, aw1, ab1, aw2, ab2)


def _split_edge_weights(p, pre):
    w1 = p[pre + "_W1"]
    return (w1[:2 * D].astype(BF16),
            w1[2 * D:].astype(BF16), p[pre + "_b1"].reshape(1, -1),
            p[pre + "_W2"].astype(BF16), p[pre + "_b2"].reshape(1, -1))


def _layer_weights(p, pre):
    cw = _split_edge_weights(p, pre + "_chi") + (
        p[pre + "_chi_W3"].astype(BF16), p[pre + "_chi_b3"].reshape(1, -1))
    w3 = p[pre + "_phi_W3"].astype(BF16)
    w3p = jnp.concatenate([w3, jnp.zeros((HID, MSGW - MD), BF16)], axis=1)
    b3 = p[pre + "_phi_b3"]
    b3p = jnp.concatenate(
        [b3, jnp.ones((1,), F32), jnp.zeros((MSGW - MD - 1,), F32)]
    ).reshape(1, -1)
    pw = _split_edge_weights(p, pre + "_phi") + (w3p, b3p)
    wg1 = p[pre + "_gamma_W1"]
    gw = (wg1[:128].astype(BF16), wg1[128:D].astype(BF16),
          wg1[D:MD].astype(BF16), wg1[MD:].astype(BF16),
          p[pre + "_gamma_b1"].reshape(1, -1),
          p[pre + "_gamma_W2"].astype(BF16),
          p[pre + "_gamma_b2"].reshape(1, -1),
          p[pre + "_gamma_W3"].astype(BF16),
          p[pre + "_gamma_b3"].reshape(1, -1))
    return cw, pw, gw


def kernel(x, edge_index, edge_attr, node_batch, edge_batch, ptr, params):
    src = edge_index[0]
    dst = edge_index[1]
    src_p = jnp.concatenate([src, jnp.zeros((PADN,), src.dtype)])
    dst_g = jnp.concatenate([dst, jnp.zeros((PADN,), dst.dtype)])
    dst_s = jnp.concatenate([dst, jnp.full((PADN,), N, dst.dtype)])
    ea_p = jnp.concatenate([edge_attr, jnp.zeros((PADN, DE), F32)])
    halves = [(src_p[:EH], dst_g[:EH], dst_s[:EH], ea_p[:EH]),
              (src_p[EH:], dst_g[EH:], dst_s[EH:], ea_p[EH:])]
    ebf = jnp.concatenate(
        [edge_batch, jnp.full((PADN,), G, edge_batch.dtype)]
    ).astype(F32).reshape(E_PAD, 1)
    nbf = node_batch.astype(F32).reshape(N, 1)
    ptr8 = ptr[:G].astype(F32).reshape(1, G)

    def layer(xb2d, ea_halves, pre, out_dtype):
        cw, pw, gw = _layer_weights(params, pre)
        xi = jax.lax.bitcast_convert_type(
            jnp.stack([xb2d[:, :128], xb2d[:, 128:]], axis=-1), jnp.int32)
        xi = jnp.concatenate([xi, jnp.zeros((NSP - N, 128), jnp.int32)])
        eos, msgs = [], []
        for (srch, dstgh, _, _), eah in zip(halves, ea_halves):
            xs, xd = _sc_gather(xi, srch, dstgh)
            eo, msg = _edge_mlp(xs, xd, eah, cw, pw)
            eos.append(eo)
            msgs.append(msg)
        slab_halves = []
        for (_, _, dstsh, _), msg in zip(halves, msgs):
            raws = _sc_scatter(msg, dstsh)
            slab_halves.append(
                [jnp.concatenate([r[:NH], r[SPROWS:SPROWS + NH]])
                 for r in raws])
        xn = _gamma_mlp(slab_halves[0], slab_halves[1], xb2d, gw, out_dtype)
        return eos, xn

    ea1h, x1b = layer(x.astype(BF16), [halves[0][3], halves[1][3]],
                      "l1", BF16)
    ea2h, x2 = layer(x1b, ea1h, "l2", F32)
    ea2 = jnp.concatenate(ea2h)

    coeff = _pooling(x2, nbf, ptr8, ea2, ebf,
                     params["alpha_W1"], params["alpha_b1"].reshape(1, -1),
                     params["alpha_W2"], params["alpha_b2"].reshape(1, -1))
    return ea2[:E], x2, coeff


# speedup vs baseline: 1.0067x; 1.0067x over previous
import functools

import jax
import jax.numpy as jnp
from jax import lax
from jax.experimental import pallas as pl
from jax.experimental.pallas import tpu as pltpu
from jax.experimental.pallas import tpu_sc as plsc

N = 10000
E = 160000
D = 256
DE = 16
G = 8
CC = 2 * D + DE
MD = CC // 2
HID = 2 * CC
GI = D + MD
GH = 2 * GI
MSGW = 384

E_PAD = 163840
PADN = E_PAD - E
EH = E_PAD // 2

B_PER_W = EH // 32
G_CH = 64
G_NCH = B_PER_W // G_CH

EPT = EH // 16
S_CH = 128
S_NCH = EPT // S_CH
NH = N // 2
SPROWS = 5120
ZROWS = SPROWS // 16

BE = 1024
NB = 1000
PN = 1000
PE = E_PAD // 10

_MESH = plsc.VectorSubcoreMesh(core_axis_name="c", subcore_axis_name="s",
                               num_cores=2, num_subcores=16)

F32 = jnp.float32
BF16 = jnp.bfloat16


G_NPAIR = G_NCH // 2


NSP = 10240


def _gather_body(x_hbm, src_hbm, dst_hbm, xs_hbm, xd_hbm, xsp, idx_s, idx_d,
                 bsa, bsb, bda, bdb,
                 sem_sa, sem_sb, sem_da, sem_db,
                 sem_osa, sem_osb, sem_oda, sem_odb):
    wid = lax.axis_index("s") * 2 + lax.axis_index("c")
    s = lax.axis_index("s")
    base = wid * B_PER_W

    pltpu.sync_copy(x_hbm.at[pl.ds(s * (NSP // 16), NSP // 16)],
                    xsp.at[pl.ds(s * (NSP // 16), NSP // 16)])
    pltpu.sync_copy(src_hbm.at[pl.ds(base, B_PER_W)], idx_s)
    pltpu.sync_copy(dst_hbm.at[pl.ds(base, B_PER_W)], idx_d)
    plsc.subcore_barrier()

    def stream(idx_v, out_hbm, buf_a, buf_b, sem_a, sem_b, sem_oa, sem_ob):
        def g_src(c):
            return xsp.at[idx_v.at[pl.ds(c * G_CH, G_CH)]]

        def o_dst(c):
            return out_hbm.at[pl.ds(base + c * G_CH, G_CH)]

        def prologue():
            pltpu.async_copy(g_src(0), buf_a, sem_a)

        def pair(p):
            c0 = 2 * p
            c1 = c0 + 1

            @pl.when(p > 0)
            def _():
                pltpu.make_async_copy(buf_b, o_dst(c1 - 2), sem_ob).wait()

            pltpu.async_copy(g_src(c1), buf_b, sem_b)
            pltpu.make_async_copy(g_src(c0), buf_a, sem_a).wait()
            pltpu.async_copy(buf_a, o_dst(c0), sem_oa)

            @pl.when(p < G_NPAIR - 1)
            def _():
                pltpu.make_async_copy(buf_a, o_dst(c0), sem_oa).wait()
                pltpu.async_copy(g_src(c0 + 2), buf_a, sem_a)

            pltpu.make_async_copy(g_src(c1), buf_b, sem_b).wait()
            pltpu.async_copy(buf_b, o_dst(c1), sem_ob)

        def epilogue():
            pltpu.make_async_copy(buf_a, o_dst(G_NCH - 2), sem_oa).wait()
            pltpu.make_async_copy(buf_b, o_dst(G_NCH - 1), sem_ob).wait()

        return prologue, pair, epilogue

    streams = [stream(idx_s, xs_hbm, bsa, bsb, sem_sa, sem_sb,
                      sem_osa, sem_osb),
               stream(idx_d, xd_hbm, bda, bdb, sem_da, sem_db,
                      sem_oda, sem_odb)]
    for pro, _, _ in streams:
        pro()

    def pair_both(p, carry):
        for _, pair, _ in streams:
            pair(p)
        return carry

    lax.fori_loop(0, G_NPAIR, pair_both, 0)
    for _, _, epi in streams:
        epi()


def _sc_gather(xi, src_p, dst_p):
    fn = pl.kernel(
        _gather_body,
        out_type=[jax.ShapeDtypeStruct((EH, 128), jnp.int32),
                  jax.ShapeDtypeStruct((EH, 128), jnp.int32)],
        mesh=_MESH,
        scratch_types=[pltpu.VMEM_SHARED((NSP, 128), jnp.int32),
                       pltpu.VMEM((B_PER_W,), jnp.int32),
                       pltpu.VMEM((B_PER_W,), jnp.int32)]
        + [pltpu.VMEM((G_CH, 128), jnp.int32)] * 4
        + [pltpu.SemaphoreType.DMA] * 8,
    )
    return fn(xi, src_p, dst_p)


S_NHALF = S_NCH // 2
S_NPAIR = S_NHALF // 2


def _make_scatter_body(width):
    def body(msg_hbm, dst_hbm, zrow_hbm, out0_hbm, out1_hbm, out2_hbm,
             shared, idx_all, idx2d, buf_a, buf_b, buf_c, buf_d,
             sem_a, sem_b, sem_c, sem_d, sem_sa, sem_sb, sem_sc, sem_sd):
        c = lax.axis_index("c")
        s = lax.axis_index("s")
        base_node = c * NH
        tbase = s * EPT

        pltpu.sync_copy(dst_hbm.at[pl.ds(tbase, EPT)], idx_all)

        def grp(k, carry):
            v = idx_all[pl.ds(k * 16, 16)]
            inr = (v >= base_node) & (v < base_node + NH)
            lv = jnp.where(inr, v - base_node, NH)
            idx2d[k // 8, pl.ds((k % 8) * 16, 16)] = lv
            return carry

        lax.fori_loop(0, EPT // 16, grp, 0)

        def stream(col0, off, buf_a, buf_b, sem_a, sem_b, sem_sa, sem_sb):
            def m_src(cc):
                return msg_hbm.at[pl.ds(tbase + (off + cc) * S_CH, S_CH),
                                  pl.ds(col0, width)]

            def sc_dst(cc):
                return shared.at[idx2d.at[off + cc]]

            def prologue():
                pltpu.async_copy(m_src(0), buf_a, sem_a)

            def pair(p):
                c0 = 2 * p
                c1 = c0 + 1

                @pl.when(p > 0)
                def _():
                    pltpu.make_async_copy(buf_b, sc_dst(c1 - 2),
                                          sem_sb).wait()

                pltpu.async_copy(m_src(c1), buf_b, sem_b)
                pltpu.make_async_copy(m_src(c0), buf_a, sem_a).wait()
                pltpu.async_copy(buf_a, sc_dst(c0), sem_sa, add=True)

                @pl.when(p < S_NPAIR - 1)
                def _():
                    pltpu.make_async_copy(buf_a, sc_dst(c0), sem_sa).wait()
                    pltpu.async_copy(m_src(c0 + 2), buf_a, sem_a)

                pltpu.make_async_copy(m_src(c1), buf_b, sem_b).wait()
                pltpu.async_copy(buf_b, sc_dst(c1), sem_sb, add=True)

            def epilogue():
                pltpu.make_async_copy(buf_a, sc_dst(S_NHALF - 2),
                                      sem_sa).wait()
                pltpu.make_async_copy(buf_b, sc_dst(S_NHALF - 1),
                                      sem_sb).wait()

            return prologue, pair, epilogue

        for col0, out_hbm in ((0, out0_hbm), (128, out1_hbm),
                              (256, out2_hbm)):
            pltpu.sync_copy(zrow_hbm, shared.at[pl.ds(s * ZROWS, ZROWS)])
            plsc.subcore_barrier()

            streams = [stream(col0, 0, buf_a, buf_b, sem_a, sem_b,
                              sem_sa, sem_sb),
                       stream(col0, S_NHALF, buf_c, buf_d, sem_c, sem_d,
                              sem_sc, sem_sd)]
            for pro, _, _ in streams:
                pro()

            def pair_both(p, carry):
                for _, pair, _ in streams:
                    pair(p)
                return carry

            lax.fori_loop(0, S_NPAIR, pair_both, 0)
            for _, _, epi in streams:
                epi()
            plsc.subcore_barrier()

            pltpu.sync_copy(shared.at[pl.ds(s * ZROWS, ZROWS)],
                            out_hbm.at[pl.ds(c * SPROWS + s * ZROWS,
                                             ZROWS)])

    return body


def _sc_scatter(msg, dst_s):
    width = 128
    zrow = jnp.zeros((ZROWS, width), F32)
    fn = pl.kernel(
        _make_scatter_body(width),
        out_type=[jax.ShapeDtypeStruct((2 * SPROWS, width), F32)] * 3,
        mesh=_MESH,
        scratch_types=[pltpu.VMEM_SHARED((SPROWS, width), F32),
                       pltpu.VMEM((EPT,), jnp.int32),
                       pltpu.VMEM((S_NCH, S_CH), jnp.int32)]
        + [pltpu.VMEM((S_CH, width), F32)] * 4
        + [pltpu.SemaphoreType.DMA] * 8,
    )
    return fn(msg, dst_s, zrow)


def _edge_mlp_body(xs_ref, xd_ref, ea_ref,
                   c1x, c1c, cb1, c2, cb2, c3, cb3,
                   p1x, p1c, pb1, p2, pb2, p3, pb3,
                   eo_ref, msg_ref):
    def unpack(v):
        lo = jax.lax.bitcast_convert_type(v << 16, F32).astype(BF16)
        hi = jax.lax.bitcast_convert_type(
            v & jnp.int32(-65536), F32).astype(BF16)
        return jnp.concatenate([lo, hi], axis=1)

    xs = unpack(xs_ref[...])
    xd = unpack(xd_ref[...])
    ea = ea_ref[...].astype(BF16)

    def dot(a, b):
        return jnp.dot(a, b, preferred_element_type=F32)

    def xdots(w_ref):
        return dot(xs, w_ref[0:256]) + dot(xd, w_ref[256:512])

    h = xdots(c1x) + dot(ea, c1c[...]) + cb1[...]
    h = jnp.maximum(h, 0.0).astype(BF16)
    h = dot(h, c2[...]) + cb2[...]
    h = jnp.maximum(h, 0.0).astype(BF16)
    eo = dot(h, c3[...]) + cb3[...]
    eo_ref[...] = eo

    g = xdots(p1x) + dot(eo.astype(BF16), p1c[...]) + pb1[...]
    g = jnp.maximum(g, 0.0).astype(BF16)
    g = dot(g, p2[...]) + pb2[...]
    g = jnp.maximum(g, 0.0).astype(BF16)
    msg_ref[...] = dot(g, p3[...]) + pb3[...]


def _edge_mlp(xs, xd, ea, cw, pw):
    grid = (EH // BE,)

    def blk(r, ccols):
        return pl.BlockSpec((r, ccols), lambda i: (i, 0))

    def full(a):
        return pl.BlockSpec(a.shape, lambda i: tuple(0 for _ in a.shape))

    ins = [blk(BE, 128), blk(BE, 128), blk(BE, DE)]
    args = [xs, xd, ea]
    for w in cw + pw:
        ins.append(full(w))
        args.append(w)
    out = pl.pallas_call(
        _edge_mlp_body,
        grid=grid,
        in_specs=ins,
        out_specs=[blk(BE, DE), blk(BE, MSGW)],
        out_shape=[jax.ShapeDtypeStruct((EH, DE), F32),
                   jax.ShapeDtypeStruct((EH, MSGW), F32)],
    )(*args)
    return out


def _gamma_body(s0a_ref, s0b_ref, s1a_ref, s1b_ref, s2a_ref, s2b_ref, x_ref,
                g1a, g1b, g1c, g1d, gb1, g2, gb2, g3, gb3, out_ref):
    s2 = s2a_ref[...] + s2b_ref[...]
    inv = 1.0 / jnp.maximum(s2[:, MD - D:MD - D + 1], 1.0)
    a0 = ((s0a_ref[...] + s0b_ref[...]) * inv).astype(BF16)
    a1 = ((s1a_ref[...] + s1b_ref[...]) * inv).astype(BF16)
    a2 = (s2[:, :MD - D] * inv).astype(BF16)
    x = x_ref[...].astype(BF16)

    def dot(a, b):
        return jnp.dot(a, b[...], preferred_element_type=F32)

    h = (dot(a0, g1a) + dot(a1, g1b) + dot(a2, g1c) + dot(x, g1d)
         + gb1[...])
    h = jnp.maximum(h, 0.0).astype(BF16)
    h = dot(h, g2) + gb2[...]
    h = jnp.maximum(h, 0.0).astype(BF16)
    out_ref[...] = (dot(h, g3) + gb3[...]).astype(out_ref.dtype)


def _gamma_mlp(slabs_a, slabs_b, xf, gw, out_dtype):
    grid = (N // NB,)

    def blk(r, ccols):
        return pl.BlockSpec((r, ccols), lambda i: (i, 0))

    def full(a):
        return pl.BlockSpec(a.shape, lambda i: (0, 0))

    ins = [blk(NB, 128)] * 6 + [blk(NB, D)]
    args = [slabs_a[0], slabs_b[0], slabs_a[1], slabs_b[1],
            slabs_a[2], slabs_b[2], xf]
    for w in gw:
        ins.append(full(w))
        args.append(w)
    return pl.pallas_call(
        _gamma_body,
        grid=grid,
        in_specs=ins,
        out_specs=blk(NB, D),
        out_shape=jax.ShapeDtypeStruct((N, D), out_dtype),
    )(*args)


def _pool_body(x2_ref, nbf_ref, ptr_ref, ea_ref, ebf_ref,
               aw1, ab1, aw2, ab2, coeff_ref, np_acc, ep_acc, ec_acc):
    i = pl.program_id(0)

    @pl.when(i == 0)
    def _():
        np_acc[...] = jnp.zeros_like(np_acc)
        ep_acc[...] = jnp.zeros_like(ep_acc)
        ec_acc[...] = jnp.zeros_like(ec_acc)

    hi = jax.lax.Precision.HIGHEST
    nbf = nbf_ref[...]
    gids = lax.broadcasted_iota(jnp.int32, (PN, G), 1).astype(F32)
    onehot = (nbf == gids).astype(F32)
    ptrsel = jnp.sum(onehot * ptr_ref[...], axis=1, keepdims=True)
    rowid = (lax.broadcasted_iota(jnp.int32, (PN, 1), 0)
             + i * PN).astype(F32)
    weight = rowid - ptrsel + 1.0
    xw = x2_ref[...] * weight
    np_acc[...] += lax.dot_general(onehot, xw, (((0,), (0,)), ((), ())),
                                   precision=hi)
    ebf = ebf_ref[...]
    gide = lax.broadcasted_iota(jnp.int32, (PE, G), 1).astype(F32)
    onehote = (ebf == gide).astype(F32)
    ep_acc[...] += lax.dot_general(onehote, ea_ref[...],
                                   (((0,), (0,)), ((), ())), precision=hi)
    ones = jnp.ones((PE, 1), F32)
    ec_acc[...] += lax.dot_general(onehote, ones, (((0,), (0,)), ((), ())),
                                   precision=hi)

    @pl.when(i == pl.num_programs(0) - 1)
    def _():
        epool = ep_acc[...] / jnp.maximum(ec_acc[...], 1.0)
        pc = jnp.concatenate([np_acc[...], epool], axis=1)
        h = jnp.maximum(
            jnp.dot(pc, aw1[...], precision=hi,
                    preferred_element_type=F32) + ab1[...], 0.0)
        coeff_ref[...] = jnp.dot(h, aw2[...], precision=hi,
                                 preferred_element_type=F32) + ab2[...]


def _pooling(x2, nbf, ptr8, ea2, ebf, aw1, ab1, aw2, ab2):
    grid = (N // PN,)

    def blk(r, ccols):
        return pl.BlockSpec((r, ccols), lambda i: (i, 0))

    def full(a):
        return pl.BlockSpec(a.shape, lambda i: (0, 0))

    return pl.pallas_call(
        _pool_body,
        grid=grid,
        in_specs=[blk(PN, D), blk(PN, 1), full(ptr8), blk(PE, DE), blk(PE, 1),
                  full(aw1), full(ab1), full(aw2), full(ab2)],
        out_specs=full(jnp.zeros((G, 5))),
        out_shape=jax.ShapeDtypeStruct((G, 5), F32),
        scratch_shapes=[pltpu.VMEM((G, D), F32), pltpu.VMEM((G, DE), F32),
                        pltpu.VMEM((G, 1), F32)],
    )(x2, nbf, ptr8, ea2, ebf, aw1, ab1, aw2, ab2)


def _split_edge_weights(p, pre):
    w1 = p[pre + "_W1"]
    return (w1[:2 * D].astype(BF16),
            w1[2 * D:].astype(BF16), p[pre + "_b1"].reshape(1, -1),
            p[pre + "_W2"].astype(BF16), p[pre + "_b2"].reshape(1, -1))


def _layer_weights(p, pre):
    cw = _split_edge_weights(p, pre + "_chi") + (
        p[pre + "_chi_W3"].astype(BF16), p[pre + "_chi_b3"].reshape(1, -1))
    w3 = p[pre + "_phi_W3"].astype(BF16)
    w3p = jnp.concatenate([w3, jnp.zeros((HID, MSGW - MD), BF16)], axis=1)
    b3 = p[pre + "_phi_b3"]
    b3p = jnp.concatenate(
        [b3, jnp.ones((1,), F32), jnp.zeros((MSGW - MD - 1,), F32)]
    ).reshape(1, -1)
    pw = _split_edge_weights(p, pre + "_phi") + (w3p, b3p)
    wg1 = p[pre + "_gamma_W1"]
    gw = (wg1[:128].astype(BF16), wg1[128:D].astype(BF16),
          wg1[D:MD].astype(BF16), wg1[MD:].astype(BF16),
          p[pre + "_gamma_b1"].reshape(1, -1),
          p[pre + "_gamma_W2"].astype(BF16),
          p[pre + "_gamma_b2"].reshape(1, -1),
          p[pre + "_gamma_W3"].astype(BF16),
          p[pre + "_gamma_b3"].reshape(1, -1))
    return cw, pw, gw


def kernel(x, edge_index, edge_attr, node_batch, edge_batch, ptr, params):
    src = edge_index[0]
    dst = edge_index[1]
    src_p = jnp.concatenate([src, jnp.zeros((PADN,), src.dtype)])
    dst_g = jnp.concatenate([dst, jnp.zeros((PADN,), dst.dtype)])
    dst_s = jnp.concatenate([dst, jnp.full((PADN,), N, dst.dtype)])
    ea_p = jnp.concatenate([edge_attr, jnp.zeros((PADN, DE), F32)])
    halves = [(src_p[:EH], dst_g[:EH], dst_s[:EH], ea_p[:EH]),
              (src_p[EH:], dst_g[EH:], dst_s[EH:], ea_p[EH:])]
    ebf = jnp.concatenate(
        [edge_batch, jnp.full((PADN,), G, edge_batch.dtype)]
    ).astype(F32).reshape(E_PAD, 1)
    nbf = node_batch.astype(F32).reshape(N, 1)
    ptr8 = ptr[:G].astype(F32).reshape(1, G)

    def layer(xb2d, ea_halves, pre, out_dtype):
        cw, pw, gw = _layer_weights(params, pre)
        xi = jax.lax.bitcast_convert_type(
            jnp.stack([xb2d[:, :128], xb2d[:, 128:]], axis=-1), jnp.int32)
        xi = jnp.concatenate([xi, jnp.zeros((NSP - N, 128), jnp.int32)])
        eos, msgs = [], []
        for (srch, dstgh, _, _), eah in zip(halves, ea_halves):
            xs, xd = _sc_gather(xi, srch, dstgh)
            eo, msg = _edge_mlp(xs, xd, eah, cw, pw)
            eos.append(eo)
            msgs.append(msg)
        slab_halves = []
        for (_, _, dstsh, _), msg in zip(halves, msgs):
            raws = _sc_scatter(msg, dstsh)
            slab_halves.append(
                [jnp.concatenate([r[:NH], r[SPROWS:SPROWS + NH]])
                 for r in raws])
        xn = _gamma_mlp(slab_halves[0], slab_halves[1], xb2d, gw, out_dtype)
        return eos, xn

    ea1h, x1b = layer(x.astype(BF16), [halves[0][3], halves[1][3]],
                      "l1", BF16)
    ea2h, x2 = layer(x1b, ea1h, "l2", F32)
    ea2 = jnp.concatenate(ea2h)

    coeff = _pooling(x2, nbf, ptr8, ea2, ebf,
                     params["alpha_W1"], params["alpha_b1"].reshape(1, -1),
                     params["alpha_W2"], params["alpha_b2"].reshape(1, -1))
    return ea2[:E], x2, coeff
